# baseline (device time: 13029 ns/iter reference)
import jax
import jax.numpy as jnp
from jax import lax
from jax.experimental import pallas as pl
from jax.experimental.pallas import tpu as pltpu

C = 4


def kernel(partial, gamma):
    _, m2, d = partial.shape
    m = m2 // 2
    q = m // 2
    qc = q // C
    p = partial.reshape(m2, d)
    g = gamma.reshape(1, d)

    def body(p_ref, g_ref, out_ref,
             xs_ref, xr_ref, ys_ref, yr_ref,
             xs_sems, xr_sems, ys_sems, yr_sems):
        my_x = lax.axis_index("x")
        my_y = lax.axis_index("y")
        peer_x = 1 - my_x
        peer_y = 1 - my_y

        barrier_sem = pltpu.get_barrier_semaphore()
        pl.semaphore_signal(
            barrier_sem, inc=1,
            device_id=(peer_x, my_y), device_id_type=pl.DeviceIdType.MESH,
        )
        pl.semaphore_signal(
            barrier_sem, inc=1,
            device_id=(my_x, peer_y), device_id_type=pl.DeviceIdType.MESH,
        )
        pl.semaphore_wait(barrier_sem, 2)

        my_q0 = my_x * m + my_y * q
        peer_q0 = peer_x * m + my_y * q

        x_rdmas = []
        for c in range(C):
            sl = pl.ds(c * qc, qc)
            xs_ref[sl, :] = p_ref[pl.ds(peer_q0 + c * qc, qc), :].astype(
                jnp.bfloat16
            )
            rdma = pltpu.make_async_remote_copy(
                src_ref=xs_ref.at[sl],
                dst_ref=xr_ref.at[sl],
                send_sem=xs_sems.at[c],
                recv_sem=xr_sems.at[c],
                device_id=(peer_x, my_y),
                device_id_type=pl.DeviceIdType.MESH,
            )
            rdma.start()
            x_rdmas.append(rdma)

        y_rdmas = []
        for c in range(C):
            sl = pl.ds(c * qc, qc)
            x_rdmas[c].wait_recv()
            y_c = p_ref[pl.ds(my_q0 + c * qc, qc), :] + xr_ref[sl, :].astype(
                jnp.float32
            )
            ms = jnp.mean(y_c * y_c, axis=-1, keepdims=True)
            o_c = y_c * lax.rsqrt(ms + 1e-6) * g_ref[:, :]
            out_ref[pl.ds(my_y * q + c * qc, qc), :] = o_c
            ys_ref[sl, :] = o_c.astype(jnp.bfloat16)
            rdma = pltpu.make_async_remote_copy(
                src_ref=ys_ref.at[sl],
                dst_ref=yr_ref.at[sl],
                send_sem=ys_sems.at[c],
                recv_sem=yr_sems.at[c],
                device_id=(my_x, peer_y),
                device_id_type=pl.DeviceIdType.MESH,
            )
            rdma.start()
            y_rdmas.append(rdma)

        for c in range(C):
            sl = pl.ds(c * qc, qc)
            y_rdmas[c].wait_recv()
            out_ref[pl.ds(peer_y * q + c * qc, qc), :] = yr_ref[sl, :].astype(
                jnp.float32
            )

        for c in range(C):
            x_rdmas[c].wait_send()
            y_rdmas[c].wait_send()

    return pl.pallas_call(
        body,
        out_shape=jax.ShapeDtypeStruct((m, d), jnp.float32),
        in_specs=[
            pl.BlockSpec(memory_space=pltpu.VMEM),
            pl.BlockSpec(memory_space=pltpu.VMEM),
        ],
        out_specs=pl.BlockSpec(memory_space=pltpu.VMEM),
        scratch_shapes=[
            pltpu.VMEM((q, d), jnp.bfloat16),
            pltpu.VMEM((q, d), jnp.bfloat16),
            pltpu.VMEM((q, d), jnp.bfloat16),
            pltpu.VMEM((q, d), jnp.bfloat16),
            pltpu.SemaphoreType.DMA((C,)),
            pltpu.SemaphoreType.DMA((C,)),
            pltpu.SemaphoreType.DMA((C,)),
            pltpu.SemaphoreType.DMA((C,)),
        ],
        compiler_params=pltpu.CompilerParams(collective_id=0),
    )(p, g)


# device time: 12564 ns/iter; 1.0370x vs baseline; 1.0370x over previous
import jax
import jax.numpy as jnp
from jax import lax
from jax.experimental import pallas as pl
from jax.experimental.pallas import tpu as pltpu

C = 4


def kernel(partial, gamma):
    _, m2, d = partial.shape
    m = m2 // 2
    q = m // 2
    qc = q // C
    p = partial.reshape(m2, d).astype(jnp.bfloat16)
    g = gamma.reshape(1, d)

    def body(p_ref, g_ref, out_ref,
             xr_ref, ys_ref, yr_ref,
             xs_sems, xr_sems, ys_sems, yr_sems):
        my_x = lax.axis_index("x")
        my_y = lax.axis_index("y")
        peer_x = 1 - my_x
        peer_y = 1 - my_y

        barrier_sem = pltpu.get_barrier_semaphore()
        pl.semaphore_signal(
            barrier_sem, inc=1,
            device_id=(peer_x, my_y), device_id_type=pl.DeviceIdType.MESH,
        )
        pl.semaphore_signal(
            barrier_sem, inc=1,
            device_id=(my_x, peer_y), device_id_type=pl.DeviceIdType.MESH,
        )
        pl.semaphore_wait(barrier_sem, 2)

        my_q0 = my_x * m + my_y * q
        peer_q0 = peer_x * m + my_y * q

        x_rdmas = []
        for c in range(C):
            sl = pl.ds(c * qc, qc)
            rdma = pltpu.make_async_remote_copy(
                src_ref=p_ref.at[pl.ds(peer_q0 + c * qc, qc), :],
                dst_ref=xr_ref.at[sl],
                send_sem=xs_sems.at[c],
                recv_sem=xr_sems.at[c],
                device_id=(peer_x, my_y),
                device_id_type=pl.DeviceIdType.MESH,
            )
            rdma.start()
            x_rdmas.append(rdma)

        y_rdmas = []
        for c in range(C):
            sl = pl.ds(c * qc, qc)
            x_rdmas[c].wait_recv()
            y_c = (
                p_ref[pl.ds(my_q0 + c * qc, qc), :].astype(jnp.float32)
                + xr_ref[sl, :].astype(jnp.float32)
            )
            ms = jnp.mean(y_c * y_c, axis=-1, keepdims=True)
            o_c = y_c * lax.rsqrt(ms + 1e-6) * g_ref[:, :]
            out_ref[pl.ds(my_y * q + c * qc, qc), :] = o_c
            ys_ref[sl, :] = o_c.astype(jnp.bfloat16)
            rdma = pltpu.make_async_remote_copy(
                src_ref=ys_ref.at[sl],
                dst_ref=yr_ref.at[sl],
                send_sem=ys_sems.at[c],
                recv_sem=yr_sems.at[c],
                device_id=(my_x, peer_y),
                device_id_type=pl.DeviceIdType.MESH,
            )
            rdma.start()
            y_rdmas.append(rdma)

        for c in range(C):
            sl = pl.ds(c * qc, qc)
            y_rdmas[c].wait_recv()
            out_ref[pl.ds(peer_y * q + c * qc, qc), :] = yr_ref[sl, :].astype(
                jnp.float32
            )

        for c in range(C):
            x_rdmas[c].wait_send()
            y_rdmas[c].wait_send()

    return pl.pallas_call(
        body,
        out_shape=jax.ShapeDtypeStruct((m, d), jnp.float32),
        in_specs=[
            pl.BlockSpec(memory_space=pltpu.VMEM),
            pl.BlockSpec(memory_space=pltpu.VMEM),
        ],
        out_specs=pl.BlockSpec(memory_space=pltpu.VMEM),
        scratch_shapes=[
            pltpu.VMEM((q, d), jnp.bfloat16),
            pltpu.VMEM((q, d), jnp.bfloat16),
            pltpu.VMEM((q, d), jnp.bfloat16),
            pltpu.SemaphoreType.DMA((C,)),
            pltpu.SemaphoreType.DMA((C,)),
            pltpu.SemaphoreType.DMA((C,)),
            pltpu.SemaphoreType.DMA((C,)),
        ],
        compiler_params=pltpu.CompilerParams(collective_id=0),
    )(p, g)
